# Initial kernel scaffold; baseline (speedup 1.0000x reference)
import jax
import jax.numpy as jnp
from jax.experimental import pallas as pl


def kernel(inputs, codebook):
    raise NotImplementedError("write your pallas kernel here")


# speedup vs baseline: 2.5786x; 2.5786x over previous
import functools

import jax
import jax.numpy as jnp
from jax import lax
from jax.experimental import pallas as pl

_BIN = 512
_K = 16
_DMULT = 0.1


def _block_body(parts_ref, perm_ref, dst_ref, val_ref):
    x = parts_ref[0]
    g = perm_ref[0, 0, 0]
    na = jnp.sum(x * x, axis=1, keepdims=True)
    gram = lax.dot_general(x, x, (((1,), (1,)), ((), ())),
                           preferred_element_type=jnp.float32)
    dsq = na - 2.0 * gram + jnp.transpose(na)
    dm = jnp.exp(-_DMULT * jnp.sqrt(jnp.maximum(dsq, 1e-6)))

    colj = lax.broadcasted_iota(jnp.int32, (_BIN, _BIN), 1)
    comp_base = colj * 8192 + g[None, :]
    big = jnp.int32(2 ** 30)

    dsts, vals = [], []
    cur = dm
    for _ in range(_K):
        m = jnp.max(cur, axis=1, keepdims=True)
        comp = jnp.where(cur == m, comp_base, big)
        r = jnp.min(comp, axis=1)
        selj = r // 8192
        dsts.append(r - selj * 8192)
        vals.append(m[:, 0])
        cur = jnp.where(colj == selj[:, None], -1.0, cur)

    dst = jnp.stack(dsts, axis=1)
    val = jnp.stack(vals, axis=1)

    rank = jnp.sum((dst[:, None, :] < dst[:, :, None]).astype(jnp.int32),
                   axis=2)
    sd, sv = [], []
    for u in range(_K):
        sel = rank == u
        sd.append(jnp.sum(jnp.where(sel, dst, 0), axis=1))
        sv.append(jnp.sum(jnp.where(sel, val, 0.0), axis=1))
    dst_ref[0] = jnp.stack(sd, axis=1)
    val_ref[0] = jnp.stack(sv, axis=1)


def _topk_blocks(parts, perm4, B, nbins, D):
    return pl.pallas_call(
        _block_body,
        grid=(B, nbins),
        in_specs=[
            pl.BlockSpec((1, _BIN, D), lambda b, n: (b, n, 0)),
            pl.BlockSpec((1, 1, 1, _BIN), lambda b, n: (b, n, 0, 0)),
        ],
        out_specs=[
            pl.BlockSpec((1, _BIN, _K), lambda b, n: (b, n, 0)),
            pl.BlockSpec((1, _BIN, _K), lambda b, n: (b, n, 0)),
        ],
        out_shape=[
            jax.ShapeDtypeStruct((B, nbins * _BIN, _K), jnp.int32),
            jax.ShapeDtypeStruct((B, nbins * _BIN, _K), jnp.float32),
        ],
    )(parts, perm4)


def kernel(inputs, codebook):
    B, N, D = inputs.shape
    nbins = N // _BIN
    mul = inputs @ codebook[:, : nbins // 2]
    cmul = jnp.concatenate([mul, -mul], axis=-1)
    bin_idx = jnp.argmax(cmul, axis=-1)
    perm = jnp.argsort(bin_idx, axis=-1).astype(jnp.int32)
    parts = jnp.take_along_axis(inputs, perm[..., None], axis=1)

    dst, val = _topk_blocks(parts, perm.reshape(B, nbins, 1, _BIN),
                            B, nbins, D)

    inv = jnp.argsort(perm, axis=-1)
    dstg = jnp.take_along_axis(dst, inv[..., None], axis=1).reshape(B, N * _K)
    valg = jnp.take_along_axis(val, inv[..., None], axis=1).reshape(B, N * _K)

    bids = jnp.broadcast_to(
        jnp.arange(B, dtype=jnp.int32)[:, None], (B, N * _K))
    srcs = jnp.broadcast_to(
        jnp.repeat(jnp.arange(N, dtype=jnp.int32), _K)[None, :], (B, N * _K))
    full_idx = jnp.stack([bids, srcs, dstg], axis=-1)
    return full_idx, valg
